# baseline (speedup 1.0000x reference)
import functools

import jax
import jax.numpy as jnp
from jax import lax
from jax.experimental import pallas as pl
from jax.experimental.pallas import tpu as pltpu
from jax.experimental.pallas import tpu_sc as plsc

N = 4096
C = 1000
RB = 256
NBLK = N // RB
NEG_INF = float("-inf")
POS_INF = float("inf")


def _stats_body(x_ref, lab_ref, conf_ref, hits_ref):
    x = x_ref[...]
    lab = lab_ref[0, :]
    maxv = jnp.max(x, axis=1, keepdims=True)
    s = jnp.sum(jnp.exp(x - maxv), axis=1)
    col = lax.broadcasted_iota(jnp.int32, x.shape, 1)
    am = jnp.min(jnp.where(x >= maxv, col, C), axis=1)
    conf_ref[0, :] = 1.0 / s
    hits_ref[0, :] = (am == lab).astype(jnp.float32)


def _stats(x, lab2):
    return pl.pallas_call(
        _stats_body,
        grid=(NBLK,),
        in_specs=[
            pl.BlockSpec((RB, C), lambda i: (i, 0)),
            pl.BlockSpec((1, RB), lambda i: (0, i)),
        ],
        out_specs=[
            pl.BlockSpec((1, RB), lambda i: (0, i)),
            pl.BlockSpec((1, RB), lambda i: (0, i)),
        ],
        out_shape=[
            jax.ShapeDtypeStruct((1, N), jnp.float32),
            jax.ShapeDtypeStruct((1, N), jnp.float32),
        ],
    )(x, lab2)


def _rank_body(conf_ref, rank_ref):
    i = pl.program_id(0)
    cj = conf_ref[0, :].reshape(1, N)
    ci = conf_ref[0, pl.ds(i * RB, RB)].reshape(RB, 1)
    jidx = lax.broadcasted_iota(jnp.int32, (RB, N), 1)
    iidx = i * RB + lax.broadcasted_iota(jnp.int32, (RB, N), 0)
    before = (cj < ci) | ((cj == ci) & (jidx < iidx))
    rank_ref[0, :] = jnp.sum(before.astype(jnp.int32), axis=1)


def _ranks(conf):
    return pl.pallas_call(
        _rank_body,
        grid=(NBLK,),
        in_specs=[pl.BlockSpec((1, N), lambda i: (0, 0))],
        out_specs=pl.BlockSpec((1, RB), lambda i: (0, i)),
        out_shape=jax.ShapeDtypeStruct((1, N), jnp.int32),
    )(conf)


def _sc_scatter_scan(rank, hits):
    mesh = plsc.VectorSubcoreMesh(core_axis_name="c", subcore_axis_name="s")

    @functools.partial(
        pl.kernel,
        mesh=mesh,
        compiler_params=pltpu.CompilerParams(needs_layout_passes=False),
        out_type=[
            jax.ShapeDtypeStruct((N,), jnp.float32),
            jax.ShapeDtypeStruct((N,), jnp.float32),
        ],
        scratch_types=[
            pltpu.VMEM((N,), jnp.int32),
            pltpu.VMEM((N,), jnp.float32),
            pltpu.VMEM((N,), jnp.float32),
            pltpu.VMEM((N,), jnp.float32),
        ],
    )
    def sc_kernel(rank_hbm, hits_hbm, hs_out, cinc_out, rank_v, hits_v,
                  hs_v, cs_v):
        cid = lax.axis_index("c")
        sid = lax.axis_index("s")

        @pl.when(jnp.logical_and(cid == 0, sid == 0))
        def _():
            pltpu.sync_copy(rank_hbm, rank_v)
            pltpu.sync_copy(hits_hbm, hits_v)

            def scatter_body(i, carry):
                kv = rank_v[pl.ds(i * 16, 16)]
                hv = hits_v[pl.ds(i * 16, 16)]
                plsc.store_scatter(hs_v, [kv], hv)
                return carry

            lax.fori_loop(0, N // 16, scatter_body, 0)

            def scan_body(i, carry):
                hv = hs_v[pl.ds(i * 16, 16)]
                cs = plsc.cumsum(hv) + carry
                cs_v[pl.ds(i * 16, 16)] = cs
                return carry + jnp.sum(hv)

            lax.fori_loop(0, N // 16, scan_body, jnp.float32(0.0))

            pltpu.sync_copy(hs_v, hs_out)
            pltpu.sync_copy(cs_v, cinc_out)

    return sc_kernel(rank, hits)


KT = 256


def _pava_body(cinc_ref, hs_ref, out_ref, carry_ref, acc_ref):
    b = pl.program_id(0)
    base = b * RB

    @pl.when(b == 0)
    def _():
        carry_ref[...] = jnp.full((1, N), NEG_INF, jnp.float32)

    sexc = (cinc_ref[0, pl.ds(base, RB)]
            - hs_ref[0, pl.ds(base, RB)]).reshape(RB, 1)
    rowiota = lax.broadcasted_iota(jnp.int32, (RB, KT), 0)
    coliota = lax.broadcasted_iota(jnp.int32, (RB, KT), 1)
    acc_ref[...] = jnp.full((RB, 1), POS_INF, jnp.float32)

    def tile_body(t, c):
        k0 = t * KT
        cinc_t = cinc_ref[0, pl.ds(k0, KT)].reshape(1, KT)
        kk = k0 + coliota
        jj = base + rowiota
        length = kk - jj + 1
        valid = length >= 1
        denom = jnp.where(valid, length, 1).astype(jnp.float32)
        M = jnp.where(valid, (cinc_t - sexc) / denom, NEG_INF)

        T1 = M
        for s in (1, 2, 4):
            shifted = jnp.concatenate(
                [jnp.full((s, KT), NEG_INF, jnp.float32), T1[: RB - s, :]],
                axis=0)
            T1 = jnp.maximum(T1, shifted)
        strip_last = T1.reshape(RB // 8, 8, KT)[:, 7, :]
        carry_row = carry_ref[0, pl.ds(k0, KT)].reshape(1, KT)
        P = jnp.maximum(strip_last, carry_row)
        s = 1
        while s < RB // 8:
            P = jnp.maximum(P, jnp.concatenate(
                [jnp.full((s, KT), NEG_INF, jnp.float32), P[: RB // 8 - s, :]],
                axis=0))
            s *= 2
        carry_ref[0, pl.ds(k0, KT)] = P[RB // 8 - 1, :]
        Pex = jnp.concatenate([carry_row, P[: RB // 8 - 1, :]], axis=0)
        E = jnp.broadcast_to(Pex.reshape(RB // 8, 1, KT),
                             (RB // 8, 8, KT)).reshape(RB, KT)
        T = jnp.maximum(T1, E)

        m = jnp.where(kk >= jj, T, POS_INF)
        acc_ref[...] = jnp.minimum(acc_ref[...],
                                   jnp.min(m, axis=1, keepdims=True))
        return c

    lax.fori_loop(b * RB // KT, N // KT, tile_body, 0)
    out_ref[...] = acc_ref[...]


def _pava(cinc, hs):
    return pl.pallas_call(
        _pava_body,
        grid=(NBLK,),
        in_specs=[
            pl.BlockSpec((1, N), lambda i: (0, 0)),
            pl.BlockSpec((1, N), lambda i: (0, 0)),
        ],
        out_specs=pl.BlockSpec((RB, 1), lambda i: (i, 0)),
        out_shape=jax.ShapeDtypeStruct((N, 1), jnp.float32),
        scratch_shapes=[pltpu.VMEM((1, N), jnp.float32),
                        pltpu.VMEM((RB, 1), jnp.float32)],
    )(cinc, hs)


def kernel(Simple_vector, label_list):
    lab2 = label_list.reshape(1, N)
    conf, hits = _stats(Simple_vector, lab2)
    rank = _ranks(conf)
    hits_s, cinc = _sc_scatter_scan(rank.reshape(N), hits.reshape(N))
    cali = _pava(cinc.reshape(1, N), hits_s.reshape(1, N))
    return cali.reshape(N), hits_s > jnp.float32(0.5)


__all__ = ["kernel"]


# speedup vs baseline: 1.0979x; 1.0979x over previous
import functools

import jax
import jax.numpy as jnp
from jax import lax
from jax.experimental import pallas as pl
from jax.experimental.pallas import tpu as pltpu
from jax.experimental.pallas import tpu_sc as plsc

N = 4096
C = 1000
RB = 256
NBLK = N // RB
NEG_INF = float("-inf")
POS_INF = float("inf")


def _stats_body(x_ref, lab_ref, conf_ref, hits_ref):
    x = x_ref[...]
    lab = lab_ref[0, :]
    maxv = jnp.max(x, axis=1, keepdims=True)
    s = jnp.sum(jnp.exp(x - maxv), axis=1)
    col = lax.broadcasted_iota(jnp.int32, x.shape, 1)
    am = jnp.min(jnp.where(x >= maxv, col, C), axis=1)
    conf_ref[0, :] = 1.0 / s
    hits_ref[0, :] = (am == lab).astype(jnp.float32)


SB = 512


def _stats(x, lab2):
    return pl.pallas_call(
        _stats_body,
        grid=(N // SB,),
        in_specs=[
            pl.BlockSpec((SB, C), lambda i: (i, 0)),
            pl.BlockSpec((1, SB), lambda i: (0, i)),
        ],
        out_specs=[
            pl.BlockSpec((1, SB), lambda i: (0, i)),
            pl.BlockSpec((1, SB), lambda i: (0, i)),
        ],
        out_shape=[
            jax.ShapeDtypeStruct((1, N), jnp.float32),
            jax.ShapeDtypeStruct((1, N), jnp.float32),
        ],
    )(x, lab2)


def _rank_body(conf_ref, rank_ref):
    i = pl.program_id(0)
    cj = conf_ref[0, :].reshape(1, N)
    ci = conf_ref[0, pl.ds(i * RB, RB)].reshape(RB, 1)
    jidx = lax.broadcasted_iota(jnp.int32, (RB, N), 1)
    iidx = i * RB + lax.broadcasted_iota(jnp.int32, (RB, N), 0)
    before = (cj < ci) | ((cj == ci) & (jidx < iidx))
    rank_ref[0, :] = jnp.sum(before.astype(jnp.int32), axis=1)


def _ranks(conf):
    return pl.pallas_call(
        _rank_body,
        grid=(NBLK,),
        in_specs=[pl.BlockSpec((1, N), lambda i: (0, 0))],
        out_specs=pl.BlockSpec((1, RB), lambda i: (0, i)),
        out_shape=jax.ShapeDtypeStruct((1, N), jnp.int32),
    )(conf)


def _sc_scatter_scan(rank, hits):
    mesh = plsc.VectorSubcoreMesh(core_axis_name="c", subcore_axis_name="s")

    @functools.partial(
        pl.kernel,
        mesh=mesh,
        compiler_params=pltpu.CompilerParams(needs_layout_passes=False),
        out_type=[
            jax.ShapeDtypeStruct((N,), jnp.float32),
            jax.ShapeDtypeStruct((N,), jnp.float32),
        ],
        scratch_types=[
            pltpu.VMEM((N,), jnp.int32),
            pltpu.VMEM((N,), jnp.float32),
            pltpu.VMEM((N,), jnp.float32),
            pltpu.VMEM((N,), jnp.float32),
        ],
    )
    def sc_kernel(rank_hbm, hits_hbm, hs_out, cinc_out, rank_v, hits_v,
                  hs_v, cs_v):
        cid = lax.axis_index("c")
        sid = lax.axis_index("s")

        @pl.when(jnp.logical_and(cid == 0, sid == 0))
        def _():
            pltpu.sync_copy(rank_hbm, rank_v)
            pltpu.sync_copy(hits_hbm, hits_v)

            def scatter_body(i, carry):
                kv = rank_v[pl.ds(i * 16, 16)]
                hv = hits_v[pl.ds(i * 16, 16)]
                plsc.store_scatter(hs_v, [kv], hv)
                return carry

            lax.fori_loop(0, N // 16, scatter_body, 0)

            def scan_body(i, carry):
                hv = hs_v[pl.ds(i * 16, 16)]
                cs = plsc.cumsum(hv) + carry
                cs_v[pl.ds(i * 16, 16)] = cs
                return carry + jnp.sum(hv)

            lax.fori_loop(0, N // 16, scan_body, jnp.float32(0.0))

            pltpu.sync_copy(hs_v, hs_out)
            pltpu.sync_copy(cs_v, cinc_out)

    return sc_kernel(rank, hits)


KT = 256


def _pava_body(cinc_ref, hs_ref, out_ref, carry_ref, acc_ref):
    b = pl.program_id(0)
    base = b * RB

    @pl.when(b == 0)
    def _():
        carry_ref[...] = jnp.full((1, N), NEG_INF, jnp.float32)

    sexc = (cinc_ref[0, pl.ds(base, RB)]
            - hs_ref[0, pl.ds(base, RB)]).reshape(RB, 1)
    rowiota = lax.broadcasted_iota(jnp.int32, (RB, KT), 0)
    coliota = lax.broadcasted_iota(jnp.int32, (RB, KT), 1)
    acc_ref[...] = jnp.full((RB, 1), POS_INF, jnp.float32)

    def tile_body(t, c):
        k0 = t * KT
        cinc_t = cinc_ref[0, pl.ds(k0, KT)].reshape(1, KT)
        kk = k0 + coliota
        jj = base + rowiota
        length = kk - jj + 1
        valid = length >= 1
        denom = jnp.where(valid, length, 1).astype(jnp.float32)
        M = jnp.where(valid, (cinc_t - sexc) / denom, NEG_INF)

        T = M
        s = 1
        while s < RB:
            shifted = jnp.concatenate(
                [jnp.full((s, KT), NEG_INF, jnp.float32), T[: RB - s, :]],
                axis=0)
            T = jnp.maximum(T, shifted)
            s *= 2
        T = jnp.maximum(T, carry_ref[0, pl.ds(k0, KT)].reshape(1, KT))
        carry_ref[0, pl.ds(k0, KT)] = T[RB - 1, :]

        m = jnp.where(kk >= jj, T, POS_INF)
        acc_ref[...] = jnp.minimum(acc_ref[...],
                                   jnp.min(m, axis=1, keepdims=True))
        return c

    lax.fori_loop(b * RB // KT, N // KT, tile_body, 0)
    out_ref[...] = acc_ref[...]


def _pava(cinc, hs):
    return pl.pallas_call(
        _pava_body,
        grid=(NBLK,),
        in_specs=[
            pl.BlockSpec((1, N), lambda i: (0, 0)),
            pl.BlockSpec((1, N), lambda i: (0, 0)),
        ],
        out_specs=pl.BlockSpec((RB, 1), lambda i: (i, 0)),
        out_shape=jax.ShapeDtypeStruct((N, 1), jnp.float32),
        scratch_shapes=[pltpu.VMEM((1, N), jnp.float32),
                        pltpu.VMEM((RB, 1), jnp.float32)],
    )(cinc, hs)


def kernel(Simple_vector, label_list):
    lab2 = label_list.reshape(1, N)
    conf, hits = _stats(Simple_vector, lab2)
    rank = _ranks(conf)
    hits_s, cinc = _sc_scatter_scan(rank.reshape(N), hits.reshape(N))
    cali = _pava(cinc.reshape(1, N), hits_s.reshape(1, N))
    return cali.reshape(N), hits_s > jnp.float32(0.5)


__all__ = ["kernel"]
